# Initial kernel scaffold; baseline (speedup 1.0000x reference)
import jax
import jax.numpy as jnp
from jax.experimental import pallas as pl


def kernel(risk_data):
    raise NotImplementedError("write your pallas kernel here")


# speedup vs baseline: 3.8827x; 3.8827x over previous
import functools

import jax
import jax.numpy as jnp
from jax import lax
from jax.experimental import pallas as pl
from jax.experimental.pallas import tpu as pltpu
from jax.experimental.pallas import tpu_sc as plsc

N_ROWS = 16384
TABLE_ROWS = 1_000_000
BASIC = 16
LANES = 16

NUM_CORES = 2
NUM_SUBCORES = 16
NUM_WORKERS = NUM_CORES * NUM_SUBCORES
ROWS_BIG = 31256
ROWS_SMALL = 31248
ZERO_BLOCK = 1248
ZERO_STEPS = 25
ZERO_TAIL = 56
CAP = 2048
GATHER_CHUNK = 128
POS_ROWS = 31296


@jax.jit
def _scatter_table(ids, feats):
    mesh = plsc.VectorSubcoreMesh(core_axis_name="core", subcore_axis_name="subcore")

    @functools.partial(
        pl.kernel,
        out_type=jax.ShapeDtypeStruct((TABLE_ROWS, BASIC), jnp.float32),
        mesh=mesh,
        compiler_params=pltpu.CompilerParams(needs_layout_passes=False,
                                             use_tc_tiling_on_sc=False),
        scratch_types=[
            pltpu.VMEM((N_ROWS,), jnp.int32),
            pltpu.VMEM((CAP + LANES,), jnp.int32),
            pltpu.VMEM((CAP + LANES,), jnp.int32),
            pltpu.VMEM((CAP, BASIC), jnp.float32),
            pltpu.VMEM((ZERO_BLOCK, BASIC), jnp.float32),
            pltpu.VMEM((POS_ROWS,), jnp.int32),
            pltpu.VMEM((1, LANES), jnp.int32),
            pltpu.SemaphoreType.DMA,
            pltpu.SemaphoreType.DMA,
        ],
    )
    def run(ids_hbm, feats_hbm, out_hbm, ids_v, rows_l, emit_i, vals_v,
            zero_v, pos_v, cidx_v, sem_g, sem_z):
        wid = lax.axis_index("subcore") * NUM_CORES + lax.axis_index("core")
        lo = wid * ROWS_SMALL + jnp.minimum(wid, 8) * 8
        hi = lo + jnp.where(wid < 8, ROWS_BIG, ROWS_SMALL)
        iota = lax.iota(jnp.int32, LANES)
        zero_row = jnp.zeros((LANES,), jnp.float32)
        zero_row_i = jnp.zeros((LANES,), jnp.int32)

        pltpu.sync_copy(ids_hbm, ids_v)

        @pl.loop(0, (CAP + LANES) // LANES)
        def _(i):
            rows_l[pl.ds(i * LANES, LANES)] = iota + i * LANES

        @pl.loop(0, POS_ROWS // LANES)
        def _(i):
            pos_v[pl.ds(i * LANES, LANES)] = zero_row_i

        @pl.loop(0, ZERO_BLOCK)
        def _(i):
            plsc.store_scatter(zero_v, [jnp.full((LANES,), i, jnp.int32), iota],
                               zero_row)

        lo_dma = pl.multiple_of(lo, 8)
        zero_copies = [
            pltpu.async_copy(
                zero_v, out_hbm.at[pl.ds(lo_dma + s * ZERO_BLOCK, ZERO_BLOCK)],
                sem_z,
            )
            for s in range(ZERO_STEPS)
        ]
        zero_copies.append(
            pltpu.async_copy(
                zero_v.at[pl.ds(0, ZERO_TAIL)],
                out_hbm.at[pl.ds(pl.multiple_of(hi - ZERO_TAIL, 8), ZERO_TAIL)],
                sem_z,
            )
        )

        def scan_body(b, cnt):
            idv = ids_v[pl.ds(b * LANES, LANES)]
            m = (idv >= lo) & (idv < hi)
            plsc.store_compressed(rows_l.at[pl.ds(cnt, LANES)], iota + b * LANES,
                                  mask=m)
            plsc.store_compressed(emit_i.at[pl.ds(cnt, LANES)], idv, mask=m)
            return cnt + jnp.sum(m.astype(jnp.int32))

        cnt = lax.fori_loop(0, N_ROWS // LANES, scan_body, 0)

        gathers = [
            pltpu.async_copy(
                feats_hbm.at[rows_l.at[pl.ds(c * GATHER_CHUNK, GATHER_CHUNK)]],
                vals_v.at[pl.ds(c * GATHER_CHUNK, GATHER_CHUNK)],
                sem_g,
            )
            for c in range(CAP // GATHER_CHUNK)
        ]
        for g in gathers:
            g.wait()

        lane0 = iota == 0

        def dedup_body(p, nw):
            idp = emit_i[pl.ds(p, LANES)][0]
            off = idp - lo
            e = pos_v[pl.ds(off, LANES)][0]
            dup = e > 0
            slot = jnp.where(dup, e - 1, nw)
            plsc.store_scatter(pos_v, [jnp.full((LANES,), off, jnp.int32)],
                               jnp.full((LANES,), slot + 1, jnp.int32),
                               mask=lane0)
            row = plsc.load_gather(vals_v,
                                   [jnp.full((LANES,), p, jnp.int32), iota])
            val = jnp.where(iota == LANES - 1, jnp.float32(17.0), row)
            plsc.store_scatter(vals_v,
                               [jnp.full((LANES,), slot, jnp.int32), iota], val)
            plsc.store_scatter(emit_i, [jnp.full((LANES,), slot, jnp.int32)],
                               jnp.full((LANES,), idp, jnp.int32), mask=lane0)
            return nw + jnp.where(dup, 0, 1)

        nw = lax.fori_loop(0, cnt, dedup_body, 0)

        nw16 = ((nw + LANES - 1) // LANES) * LANES
        id0 = emit_i[pl.ds(0, LANES)][0]
        row0 = plsc.load_gather(vals_v, [jnp.zeros((LANES,), jnp.int32), iota])

        def pad_body(q, _):
            plsc.store_scatter(emit_i, [jnp.full((LANES,), q, jnp.int32)],
                               jnp.full((LANES,), id0, jnp.int32), mask=lane0)
            plsc.store_scatter(vals_v,
                               [jnp.full((LANES,), q, jnp.int32), iota], row0)
            return 0

        lax.fori_loop(nw, nw16, pad_body, 0)

        for z in zero_copies:
            z.wait()

        def scatter_body(t, _):
            civ = emit_i[pl.ds(t * LANES, LANES)]
            plsc.store_scatter(cidx_v, [jnp.zeros((LANES,), jnp.int32), iota],
                               civ)
            pltpu.sync_copy(vals_v.at[pl.ds(t * LANES, LANES)],
                            out_hbm.at[cidx_v.at[0]])
            return 0

        lax.fori_loop(0, nw16 // LANES, scatter_body, 0)

    return run(ids, feats)


def kernel(risk_data):
    ids = risk_data[:, 16].astype(jnp.int32)
    feats = risk_data[:, 1:17].astype(jnp.float32)
    return _scatter_table(ids, feats)


# speedup vs baseline: 17.1525x; 4.4177x over previous
import functools

import jax
import jax.numpy as jnp
from jax import lax
from jax.experimental import pallas as pl
from jax.experimental.pallas import tpu as pltpu
from jax.experimental.pallas import tpu_sc as plsc

N_ROWS = 16384
TABLE_ROWS = 1_000_000
BASIC = 16
LANES = 16

NUM_CORES = 2
NUM_SUBCORES = 16
NUM_WORKERS = NUM_CORES * NUM_SUBCORES
TILES_SMALL = 244
BLOCK = 512
NFULL = 61
CAP = 1024
GCHUNK = 128
NCHUNKS = CAP // GCHUNK
NBLK = 80
DCAP = 64


@jax.jit
def _scatter_table_t(ids, featsp):
    mesh = plsc.VectorSubcoreMesh(core_axis_name="core", subcore_axis_name="subcore")

    @functools.partial(
        pl.kernel,
        out_type=jax.ShapeDtypeStruct((BASIC, TABLE_ROWS), jnp.float32),
        mesh=mesh,
        compiler_params=pltpu.CompilerParams(needs_layout_passes=False,
                                             disable_bounds_checks=True),
        scratch_types=[
            pltpu.VMEM((N_ROWS,), jnp.int32),
            pltpu.VMEM((CAP + LANES,), jnp.int32),
            pltpu.VMEM((CAP + LANES,), jnp.int32),
            pltpu.VMEM((CAP + LANES,), jnp.int32),
            pltpu.VMEM((CAP * LANES + LANES,), jnp.float32),
            pltpu.VMEM((GCHUNK, 128), jnp.float32),
            pltpu.VMEM((BASIC, BLOCK), jnp.float32),
            pltpu.VMEM((BASIC, BLOCK), jnp.float32),
            pltpu.VMEM((NBLK,), jnp.int32),
            pltpu.VMEM((NBLK,), jnp.int32),
            pltpu.VMEM((NBLK,), jnp.int32),
            pltpu.VMEM((CAP + LANES,), jnp.int32),
            pltpu.VMEM((CAP + LANES,), jnp.int32),
            pltpu.VMEM((2 * DCAP + LANES,), jnp.int32),
            pltpu.SemaphoreType.DMA,
            pltpu.SemaphoreType.DMA,
            pltpu.SemaphoreType.DMA,
        ],
    )
    def run(ids_hbm, featsp_hbm, out_hbm, ids_v, rows_l, ids_l, rows8_l,
            vals_v, gst_v, stage_a, stage_b, bcnt_v, boff_v, wcur_v,
            sp_v, sid_v, dlist_v, sem_g, sem_a, sem_b):
        wid = lax.axis_index("subcore") * NUM_CORES + lax.axis_index("core")
        tile_lo = wid * TILES_SMALL + jnp.minimum(wid, 4)
        col_lo = pl.multiple_of(tile_lo * 128, 128)
        ntiles = jnp.where(wid < 4, TILES_SMALL + 1, TILES_SMALL)
        col_hi = col_lo + ntiles * 128
        mask_hi = jnp.where(wid == NUM_WORKERS - 1, TABLE_ROWS, col_hi)
        iota = lax.iota(jnp.int32, LANES)
        lane0 = iota == 0
        zrow = jnp.zeros((LANES,), jnp.float32)
        zrow_i = jnp.zeros((LANES,), jnp.int32)

        pltpu.sync_copy(ids_hbm, ids_v)

        @pl.loop(0, (CAP + LANES) // LANES)
        def _(i):
            pat = (iota + i * LANES) * 8
            rows_l[pl.ds(i * LANES, LANES)] = pat
            ids_l[pl.ds(i * LANES, LANES)] = zrow_i

        @pl.loop(0, NBLK // LANES)
        def _(i):
            bcnt_v[pl.ds(i * LANES, LANES)] = zrow_i

        @pl.loop(0, BLOCK)
        def _(c):
            cc = jnp.full((LANES,), c, jnp.int32)
            plsc.store_scatter(stage_a, [iota, cc], zrow)
            plsc.store_scatter(stage_b, [iota, cc], zrow)

        def scan_body(b, cnt):
            idv = ids_v[pl.ds(b * LANES, LANES)]
            m = (idv >= col_lo) & (idv < mask_hi)
            plsc.store_compressed(rows_l.at[pl.ds(cnt, LANES)],
                                  iota + b * LANES, mask=m)
            plsc.store_compressed(ids_l.at[pl.ds(cnt, LANES)], idv, mask=m)
            return jnp.minimum(cnt + jnp.sum(m.astype(jnp.int32)), CAP)

        cnt = lax.fori_loop(0, N_ROWS // LANES, scan_body, 0)

        @pl.loop(0, (CAP + LANES) // LANES)
        def _(i):
            rows8_l[pl.ds(i * LANES, LANES)] = (
                rows_l[pl.ds(i * LANES, LANES)] >> 3)

        for k in range(NCHUNKS):
            @pl.when(k * GCHUNK < cnt)
            def _():
                pltpu.async_copy(
                    featsp_hbm.at[rows8_l.at[pl.ds(k * GCHUNK, GCHUNK)]],
                    gst_v, sem_g).wait()
                nk = jnp.minimum(cnt - k * GCHUNK, GCHUNK)

                def extract_body(j, _):
                    p = k * GCHUNK + j
                    praw = rows_l[pl.ds(p, LANES)][0]
                    sid = ids_l[pl.ds(p, LANES)][0]
                    sub = praw & 7
                    val = plsc.load_gather(
                        gst_v, [jnp.full((LANES,), j, jnp.int32),
                                sub * LANES + iota])
                    val = jnp.where(iota == LANES - 1, jnp.float32(17.0), val)
                    plsc.store_scatter(vals_v, [p * LANES + iota], val)
                    blk = (sid - col_lo) >> 9
                    c = bcnt_v[pl.ds(blk, LANES)][0]
                    plsc.store_scatter(bcnt_v, [jnp.full((LANES,), blk, jnp.int32)],
                                       jnp.full((LANES,), c + 1, jnp.int32),
                                       mask=lane0)
                    return 0

                lax.fori_loop(0, nk, extract_body, 0)

        def prefix_body(b, run):
            c = bcnt_v[pl.ds(b, LANES)][0]
            rv = jnp.full((LANES,), run, jnp.int32)
            bv = jnp.full((LANES,), b, jnp.int32)
            plsc.store_scatter(boff_v, [bv], rv, mask=lane0)
            plsc.store_scatter(wcur_v, [bv], rv, mask=lane0)
            return run + c

        lax.fori_loop(0, NBLK, prefix_body, 0)

        def place_body(p, _):
            sid = ids_l[pl.ds(p, LANES)][0]
            blk = (sid - col_lo) >> 9
            pos = wcur_v[pl.ds(blk, LANES)][0]
            blkv = jnp.full((LANES,), blk, jnp.int32)
            posv = jnp.full((LANES,), pos, jnp.int32)
            plsc.store_scatter(wcur_v, [blkv],
                               jnp.full((LANES,), pos + 1, jnp.int32), mask=lane0)
            plsc.store_scatter(sp_v, [posv],
                               jnp.full((LANES,), p, jnp.int32), mask=lane0)
            plsc.store_scatter(sid_v, [posv],
                               jnp.full((LANES,), sid, jnp.int32), mask=lane0)
            return 0

        lax.fori_loop(0, cnt, place_body, 0)

        def rezero(stage, dslot, dcnt):
            def few(_):
                def zb(q, _):
                    cc = dlist_v[pl.ds(dslot * DCAP + q, LANES)][0]
                    plsc.store_scatter(stage,
                                       [iota, jnp.full((LANES,), cc, jnp.int32)],
                                       zrow)
                    return 0
                lax.fori_loop(0, dcnt, zb, 0)
                return 0

            def full(_):
                def zb(c, _):
                    plsc.store_scatter(stage,
                                       [iota, jnp.full((LANES,), c, jnp.int32)],
                                       zrow)
                    return 0
                lax.fori_loop(0, BLOCK, zb, 0)
                return 0

            lax.cond(dcnt <= DCAP, few, full, 0)

        def fill_block(s, stage, dslot):
            base = col_lo + s * BLOCK
            start = boff_v[pl.ds(s, LANES)][0]
            n = bcnt_v[pl.ds(s, LANES)][0]

            def wb(q, d):
                sp = sp_v[pl.ds(start + q, LANES)][0]
                sid = sid_v[pl.ds(start + q, LANES)][0]
                cc = sid - base
                val = plsc.load_gather(vals_v, [sp * LANES + iota])
                plsc.store_scatter(stage,
                                   [iota, jnp.full((LANES,), cc, jnp.int32)], val)
                plsc.store_scatter(
                    dlist_v,
                    [jnp.full((LANES,), dslot * DCAP + jnp.minimum(d, DCAP - 1),
                              jnp.int32)],
                    jnp.full((LANES,), cc, jnp.int32), mask=lane0)
                return d + 1

            return lax.fori_loop(0, n, wb, 0)

        def issue(stage, s, width, sem):
            base = pl.multiple_of(col_lo + s * BLOCK, 128)
            return pltpu.async_copy(
                stage.at[:, pl.ds(0, width)],
                out_hbm.at[:, pl.ds(base, width)], sem)

        def drain(stage, width, sem):
            pltpu.make_async_copy(
                stage.at[:, pl.ds(0, width)],
                out_hbm.at[:, pl.ds(0, width)], sem).wait()

        def pair_body(g, carry):
            da, db = carry

            def one(s, stage, sem, dslot, d):
                @pl.when(g > 0)
                def _():
                    drain(stage, BLOCK, sem)
                    rezero(stage, dslot, d)
                d2 = fill_block(s, stage, dslot)
                issue(stage, s, BLOCK, sem)
                return d2

            da = one(2 * g, stage_a, sem_a, 0, da)
            db = one(2 * g + 1, stage_b, sem_b, 1, db)
            return da, db

        da, db = lax.fori_loop(0, 30, pair_body, (0, 0))

        drain(stage_a, BLOCK, sem_a)
        rezero(stage_a, 0, da)
        fill_block(60, stage_a, 0)
        issue(stage_a, 60, BLOCK, sem_a)
        drain(stage_a, BLOCK, sem_a)

        has_tail = (wid < 4) | (wid == NUM_WORKERS - 1)

        @pl.when(has_tail)
        def _():
            drain(stage_b, BLOCK, sem_b)
            rezero(stage_b, 1, db)
            fill_block(61, stage_b, 1)
            issue(stage_b, 61, 128, sem_b)
            drain(stage_b, 128, sem_b)

        @pl.when(jnp.logical_not(has_tail))
        def _():
            drain(stage_b, BLOCK, sem_b)

    return run(ids, featsp)


def kernel(risk_data):
    ids = risk_data[:, 16].astype(jnp.int32)
    featsp = risk_data[:, 1:17].astype(jnp.float32).reshape(
        N_ROWS * BASIC // 128, 128)
    out_t = _scatter_table_t(ids, featsp)
    return out_t.T
